# baseline (speedup 1.0000x reference)
import functools

import jax
import jax.numpy as jnp
from jax import lax
from jax.experimental import pallas as pl
from jax.experimental.pallas import tpu as pltpu
from jax.experimental.pallas import tpu_sc as plsc

_NUM_ROUTES = 16
_NUM_KEEP = max(1, int(_NUM_ROUTES * (1.0 - 0.1)))
_SCALE = _NUM_ROUTES / _NUM_KEEP


def _mask_body(s_hbm, mask_hbm, s_v, mask_v):
    cid = lax.axis_index("c")
    sid = lax.axis_index("s")

    @pl.when(jnp.logical_and(cid == 0, sid == 0))
    def _():
        pltpu.sync_copy(s_hbm, s_v)
        lane = lax.broadcasted_iota(jnp.int32, (16,), 0)
        s = s_v[...]
        rank = jnp.zeros((16,), jnp.int32)
        for shift in range(1, _NUM_ROUTES):
            jv = (lane + shift) & 15
            sj = plsc.load_gather(s_v, [jv])
            beats = jnp.logical_or(sj < s, jnp.logical_and(sj == s, jv < lane))
            rank = rank + jnp.where(beats, 1, 0)
        mask_v[...] = jnp.where(rank < _NUM_KEEP, 1.0, 0.0)
        pltpu.sync_copy(mask_v, mask_hbm)


@functools.partial(
    pl.kernel,
    out_type=jax.ShapeDtypeStruct((16,), jnp.float32),
    mesh=plsc.VectorSubcoreMesh(core_axis_name="c", subcore_axis_name="s"),
    compiler_params=pltpu.CompilerParams(needs_layout_passes=False),
    scratch_types=[
        pltpu.VMEM((16,), jnp.float32),
        pltpu.VMEM((16,), jnp.float32),
    ],
)
def _route_mask_sc(s_hbm, mask_hbm, s_v, mask_v):
    _mask_body(s_hbm, mask_hbm, s_v, mask_v)


def _mul_body(s_s, x_ref, o_ref, m2d_ref):
    @pl.when(pl.program_id(0) == 0)
    def _():
        s = [s_s[i] for i in range(_NUM_ROUTES)]
        sub = lax.broadcasted_iota(jnp.int32, (16, 128), 0)
        m2d = jnp.zeros((16, 128), jnp.float32)
        for i in range(_NUM_ROUTES):
            r = jnp.int32(0)
            for j in range(_NUM_ROUTES):
                if j < i:
                    beats = jnp.logical_or(s[j] < s[i], s[j] == s[i])
                else:
                    beats = s[j] < s[i]
                r = r + jnp.where(beats, 1, 0)
            keep_scaled = jnp.where(r < _NUM_KEEP, jnp.float32(_SCALE),
                                    jnp.float32(0.0))
            m2d = jnp.where(sub == i, keep_scaled, m2d)
        m2d_ref[...] = m2d

    o_ref[...] = x_ref[...] * m2d_ref[...]


def kernel(x, importance):
    drop_weights = 1.0 / (importance + 1e-08)
    drop_weights = drop_weights / drop_weights.sum()
    noise = jax.random.uniform(jax.random.key(42), drop_weights.shape,
                               dtype=drop_weights.dtype) * 0.5
    drop_scores = drop_weights + noise

    keep_mask = _route_mask_sc(drop_scores)

    rows = 4 * 2048
    block = 1024
    x3 = x.reshape(rows, 16, 128)
    out = pl.pallas_call(
        _mul_body,
        grid_spec=pltpu.PrefetchScalarGridSpec(
            num_scalar_prefetch=1,
            grid=(rows // block,),
            in_specs=[pl.BlockSpec((block, 16, 128), lambda i, *_: (i, 0, 0))],
            out_specs=pl.BlockSpec((block, 16, 128), lambda i, *_: (i, 0, 0)),
            scratch_shapes=[pltpu.VMEM((16, 128), jnp.float32)],
        ),
        out_shape=jax.ShapeDtypeStruct((rows, 16, 128), jnp.float32),
    )(drop_scores, x3)
    return out.reshape(x.shape), keep_mask


# speedup vs baseline: 1.0298x; 1.0298x over previous
import functools

import jax
import jax.numpy as jnp
from jax import lax
from jax.experimental import pallas as pl
from jax.experimental.pallas import tpu as pltpu
from jax.experimental.pallas import tpu_sc as plsc

_NUM_ROUTES = 16
_NUM_KEEP = max(1, int(_NUM_ROUTES * (1.0 - 0.1)))
_SCALE = _NUM_ROUTES / _NUM_KEEP


def _mask_body(s_hbm, mask_hbm, s_v, mask_v):
    cid = lax.axis_index("c")
    sid = lax.axis_index("s")

    @pl.when(jnp.logical_and(cid == 0, sid == 0))
    def _():
        pltpu.sync_copy(s_hbm, s_v)
        lane = lax.broadcasted_iota(jnp.int32, (16,), 0)
        s = s_v[...]
        rank = jnp.zeros((16,), jnp.int32)
        for shift in range(1, _NUM_ROUTES):
            jv = (lane + shift) & 15
            sj = plsc.load_gather(s_v, [jv])
            beats = jnp.logical_or(sj < s, jnp.logical_and(sj == s, jv < lane))
            rank = rank + jnp.where(beats, 1, 0)
        mask_v[...] = jnp.where(rank < _NUM_KEEP, 1.0, 0.0)
        pltpu.sync_copy(mask_v, mask_hbm)


@functools.partial(
    pl.kernel,
    out_type=jax.ShapeDtypeStruct((16,), jnp.float32),
    mesh=plsc.VectorSubcoreMesh(core_axis_name="c", subcore_axis_name="s", num_cores=1),
    compiler_params=pltpu.CompilerParams(needs_layout_passes=False),
    scratch_types=[
        pltpu.VMEM((16,), jnp.float32),
        pltpu.VMEM((16,), jnp.float32),
    ],
)
def _route_mask_sc(s_hbm, mask_hbm, s_v, mask_v):
    _mask_body(s_hbm, mask_hbm, s_v, mask_v)


def _mul_body(s_s, x_ref, o_ref, m2d_ref):
    @pl.when(pl.program_id(0) == 0)
    def _():
        s = [s_s[i] for i in range(_NUM_ROUTES)]
        sub = lax.broadcasted_iota(jnp.int32, (16, 128), 0)
        m2d = jnp.zeros((16, 128), jnp.float32)
        for i in range(_NUM_ROUTES):
            r = jnp.int32(0)
            for j in range(_NUM_ROUTES):
                if j < i:
                    beats = jnp.logical_or(s[j] < s[i], s[j] == s[i])
                else:
                    beats = s[j] < s[i]
                r = r + jnp.where(beats, 1, 0)
            keep_scaled = jnp.where(r < _NUM_KEEP, jnp.float32(_SCALE),
                                    jnp.float32(0.0))
            m2d = jnp.where(sub == i, keep_scaled, m2d)
        m2d_ref[...] = m2d

    o_ref[...] = x_ref[...] * m2d_ref[...]


def kernel(x, importance):
    drop_weights = 1.0 / (importance + 1e-08)
    drop_weights = drop_weights / drop_weights.sum()
    noise = jax.random.uniform(jax.random.key(42), drop_weights.shape,
                               dtype=drop_weights.dtype) * 0.5
    drop_scores = drop_weights + noise

    keep_mask = _route_mask_sc(drop_scores)

    rows = 4 * 2048
    block = 1024
    x3 = x.reshape(rows, 16, 128)
    out = pl.pallas_call(
        _mul_body,
        grid_spec=pltpu.PrefetchScalarGridSpec(
            num_scalar_prefetch=1,
            grid=(rows // block,),
            in_specs=[pl.BlockSpec((block, 16, 128), lambda i, *_: (i, 0, 0))],
            out_specs=pl.BlockSpec((block, 16, 128), lambda i, *_: (i, 0, 0)),
            scratch_shapes=[pltpu.VMEM((16, 128), jnp.float32)],
        ),
        out_shape=jax.ShapeDtypeStruct((rows, 16, 128), jnp.float32),
    )(drop_scores, x3)
    return out.reshape(x.shape), keep_mask
